# baseline (speedup 1.0000x reference)
import functools

import jax
import jax.numpy as jnp
from jax import lax
from jax.experimental import pallas as pl
from jax.experimental.pallas import tpu as pltpu
from jax.experimental.pallas import tpu_sc as plsc

B = 25
N_PER = 400
N = B * N_PER
DEG = 16
E = N * DEG
D = 128
K = 320
PN = B * K
KK = K * K
NC = B * KK
E_PER = N_PER * DEG
HALF = N_PER // 2
LAMB = 1.0
SLOPE = 0.2

_SC_PARAMS = pltpu.CompilerParams(needs_layout_passes=False)


def _sc_build_bias(src, dst, ef, nm):
    mesh = plsc.VectorSubcoreMesh(core_axis_name="c", subcore_axis_name="s")

    @functools.partial(
        pl.kernel,
        mesh=mesh,
        out_type=jax.ShapeDtypeStruct((NC,), jnp.float32),
        scratch_types=[
            pltpu.VMEM((KK,), jnp.float32),
            pltpu.VMEM((E_PER,), jnp.int32),
            pltpu.VMEM((E_PER,), jnp.int32),
            pltpu.VMEM((E_PER,), jnp.float32),
            pltpu.VMEM((N_PER,), jnp.int32),
        ],
        compiler_params=_SC_PARAMS,
    )
    def k(src_hbm, dst_hbm, ef_hbm, nm_hbm, bias_hbm, bias_buf, s_buf, d_buf, e_buf, nm_buf):
        c = lax.axis_index("c")
        s = lax.axis_index("s")
        wid = s * 2 + c
        zero16f = jnp.zeros((16,), jnp.float32)

        @pl.when(wid < B)
        def _():
            g = wid
            pltpu.sync_copy(src_hbm.at[pl.ds(g * E_PER, E_PER)], s_buf)
            pltpu.sync_copy(dst_hbm.at[pl.ds(g * E_PER, E_PER)], d_buf)
            pltpu.sync_copy(ef_hbm.at[pl.ds(g * E_PER, E_PER)], e_buf)
            pltpu.sync_copy(nm_hbm.at[pl.ds(g * N_PER, N_PER)], nm_buf)

            def zbody(i, _):
                bias_buf[pl.ds(i * 16, 16)] = zero16f
                return 0

            lax.fori_loop(0, KK // 16, zbody, 0)

            goff = g * N_PER

            def body(i, _):
                sl = s_buf[pl.ds(i * 16, 16)] - goff
                dl = d_buf[pl.ds(i * 16, 16)] - goff
                ev = e_buf[pl.ds(i * 16, 16)]
                ms = plsc.load_gather(nm_buf, [sl])
                md = plsc.load_gather(nm_buf, [dl])
                valid = (ms >= 0) & (md >= 0)
                idx = jnp.where(valid, ms * K + md, 0)
                plsc.addupdate_scatter(bias_buf, [idx], ev, mask=valid)
                return 0

            lax.fori_loop(0, E_PER // 16, body, 0)
            pltpu.sync_copy(bias_buf, bias_hbm.at[pl.ds(g * KK, KK)])

    return k(src, dst, ef, nm)


def _dot(x, y, dims):
    return lax.dot_general(
        x, y, (dims, ((), ())),
        precision=lax.Precision.HIGHEST, preferred_element_type=jnp.float32,
    )


def _tc_topk(score3, feat_r, att2):
    def body(s_ref, f_ref, att_ref, fp_ref, perm_ref, nm_ref, av_ref, bv_ref):
        g = pl.program_id(0)
        fg = f_ref[0]
        score_col = s_ref[0]
        ii = lax.broadcasted_iota(jnp.int32, (N_PER, N_PER), 0)
        jj = lax.broadcasted_iota(jnp.int32, (N_PER, N_PER), 1)
        eye_n = jnp.where(ii == jj, 1.0, 0.0)
        score_row = _dot(score_col, eye_n, ((0,), (0,)))
        gt = score_row > score_col
        eq = score_row == score_col
        cmp_f = jnp.where(gt | (eq & (jj < ii)), 1.0, 0.0)
        ones_col = jnp.ones((N_PER, 1), jnp.float32)
        rank_col = _dot(cmp_f, ones_col, ((1,), (0,)))
        rank_row = _dot(rank_col, eye_n, ((0,), (0,)))
        rank_i = rank_row.astype(jnp.int32)
        nm_ref[0] = jnp.where(rank_i < K, rank_i, -1)
        kk = lax.broadcasted_iota(jnp.int32, (K, N_PER), 0)
        Ob_f = jnp.where(rank_i == kk, 1.0, 0.0)
        iota_col = lax.broadcasted_iota(jnp.int32, (N_PER, 1), 0).astype(jnp.float32)
        order_col = _dot(Ob_f, iota_col, ((1,), (0,)))
        ik = lax.broadcasted_iota(jnp.int32, (K, K), 0)
        jk = lax.broadcasted_iota(jnp.int32, (K, K), 1)
        eye_k = jnp.where(ik == jk, 1.0, 0.0)
        order_row = _dot(order_col, eye_k, ((0,), (0,)))
        perm_ref[0] = order_row.astype(jnp.int32) + g * N_PER
        fp_ref[0] = _dot(Ob_f, fg, ((1,), (0,)))
        sl_col = _dot(fg, att_ref[0:1, :], ((1,), (1,)))
        sr_col = _dot(fg, att_ref[1:2, :], ((1,), (1,)))
        av_ref[0] = _dot(Ob_f, sl_col, ((1,), (0,)))
        b_col = _dot(Ob_f, sr_col, ((1,), (0,)))
        bv_ref[0] = _dot(b_col, eye_k, ((0,), (0,)))

    return pl.pallas_call(
        body,
        grid=(B,),
        in_specs=[
            pl.BlockSpec((1, N_PER, 1), lambda g: (g, 0, 0)),
            pl.BlockSpec((1, N_PER, D), lambda g: (g, 0, 0)),
            pl.BlockSpec((2, D), lambda g: (0, 0)),
        ],
        out_specs=[
            pl.BlockSpec((1, K, D), lambda g: (g, 0, 0)),
            pl.BlockSpec((1, 1, K), lambda g: (g, 0, 0)),
            pl.BlockSpec((1, 1, N_PER), lambda g: (g, 0, 0)),
            pl.BlockSpec((1, K, 1), lambda g: (g, 0, 0)),
            pl.BlockSpec((1, 1, K), lambda g: (g, 0, 0)),
        ],
        out_shape=[
            jax.ShapeDtypeStruct((B, K, D), jnp.float32),
            jax.ShapeDtypeStruct((B, 1, K), jnp.int32),
            jax.ShapeDtypeStruct((B, 1, N_PER), jnp.int32),
            jax.ShapeDtypeStruct((B, K, 1), jnp.float32),
            jax.ShapeDtypeStruct((B, 1, K), jnp.float32),
        ],
    )(score3, feat_r, att2)


def _tc_softmax(a3, b3, bias3):
    def body(a_ref, b_ref, bias_ref, ws_ref, row_ref, col_ref):
        g = pl.program_id(0)
        av = a_ref[0]
        bv = b_ref[0]
        w = av + bv
        w = jnp.where(w >= 0, w, SLOPE * w)
        w = w + bias_ref[0]
        m = jnp.max(w, axis=0, keepdims=True)
        ew = jnp.exp(w - m)
        den = jnp.sum(ew, axis=0, keepdims=True)
        ws_ref[0] = ew / den
        rr = lax.broadcasted_iota(jnp.int32, (K, K), 0)
        cc = lax.broadcasted_iota(jnp.int32, (K, K), 1)
        row_ref[0] = g * K + rr
        col_ref[0] = g * K + cc

    return pl.pallas_call(
        body,
        grid=(B,),
        in_specs=[
            pl.BlockSpec((1, K, 1), lambda g: (g, 0, 0)),
            pl.BlockSpec((1, 1, K), lambda g: (g, 0, 0)),
            pl.BlockSpec((1, K, K), lambda g: (g, 0, 0)),
        ],
        out_specs=[
            pl.BlockSpec((1, K, K), lambda g: (g, 0, 0)),
            pl.BlockSpec((1, K, K), lambda g: (g, 0, 0)),
            pl.BlockSpec((1, K, K), lambda g: (g, 0, 0)),
        ],
        out_shape=[
            jax.ShapeDtypeStruct((B, K, K), jnp.float32),
            jax.ShapeDtypeStruct((B, K, K), jnp.int32),
            jax.ShapeDtypeStruct((B, K, K), jnp.int32),
        ],
    )(a3, b3, bias3)


def kernel(feat, edge_index, e_feat, att):
    src = edge_index[0]
    dst = edge_index[1]
    out_deg = jnp.maximum(jnp.zeros((N,), jnp.float32).at[src].add(1.0), 1.0)
    in_deg = jnp.maximum(jnp.zeros((N,), jnp.float32).at[dst].add(1.0), 1.0)
    src_norm = out_deg ** -0.5
    dst_norm = in_deg ** -0.5
    msg = (feat * src_norm[:, None])[src] * e_feat[:, None]
    agg = jnp.zeros((N, D), feat.dtype).at[dst].add(msg)
    f2 = feat - agg * dst_norm[:, None]
    score = jnp.sum(jnp.abs(f2), axis=1)
    feat_r = feat.reshape(B, N_PER, D)
    att2 = att.reshape(2, D)
    feat_p, perm3, nm3, a3, b3 = _tc_topk(
        score.reshape(B, N_PER, 1), feat_r, att2
    )
    bias_flat = _sc_build_bias(src, dst, e_feat, nm3.reshape(N))
    w3, row3, col3 = _tc_softmax(a3, b3, bias_flat.reshape(B, K, K))
    return (
        feat_p.reshape(PN, D),
        w3.reshape(NC),
        perm3.reshape(PN),
        row3.reshape(NC),
        col3.reshape(NC),
    )


# speedup vs baseline: 1.6515x; 1.6515x over previous
import functools

import jax
import jax.numpy as jnp
from jax import lax
from jax.experimental import pallas as pl
from jax.experimental.pallas import tpu as pltpu
from jax.experimental.pallas import tpu_sc as plsc

B = 25
N_PER = 400
N = B * N_PER
DEG = 16
E = N * DEG
D = 128
K = 320
PN = B * K
KK = K * K
NC = B * KK
E_PER = N_PER * DEG
HALF = N_PER // 2
LAMB = 1.0
SLOPE = 0.2

_SC_PARAMS = pltpu.CompilerParams(needs_layout_passes=False)


def _sc_degrees(src, dst):
    mesh = plsc.VectorSubcoreMesh(core_axis_name="c", subcore_axis_name="s")

    @functools.partial(
        pl.kernel,
        mesh=mesh,
        out_type=jax.ShapeDtypeStruct((B * 2 * N_PER,), jnp.float32),
        scratch_types=[
            pltpu.VMEM((E_PER,), jnp.int32),
            pltpu.VMEM((E_PER,), jnp.int32),
            pltpu.VMEM((2 * N_PER,), jnp.float32),
        ],
        compiler_params=_SC_PARAMS,
    )
    def k(src_hbm, dst_hbm, deg_hbm, s_buf, d_buf, deg_buf):
        c = lax.axis_index("c")
        s = lax.axis_index("s")
        wid = s * 2 + c
        zero16f = jnp.zeros((16,), jnp.float32)
        ones16 = jnp.ones((16,), jnp.float32)

        @pl.when(wid < B)
        def _():
            g = wid
            pltpu.sync_copy(src_hbm.at[pl.ds(g * E_PER, E_PER)], s_buf)
            pltpu.sync_copy(dst_hbm.at[pl.ds(g * E_PER, E_PER)], d_buf)

            def zbody(i, _):
                deg_buf[pl.ds(i * 16, 16)] = zero16f
                return 0

            lax.fori_loop(0, (2 * N_PER) // 16, zbody, 0)

            goff = g * N_PER

            def body(i, _):
                sl = s_buf[pl.ds(i * 16, 16)] - goff
                dl = d_buf[pl.ds(i * 16, 16)] - goff
                plsc.addupdate_scatter(deg_buf, [sl], ones16)
                plsc.addupdate_scatter(deg_buf, [dl + N_PER], ones16)
                return 0

            lax.fori_loop(0, E_PER // 16, body, 0)
            pltpu.sync_copy(deg_buf, deg_hbm.at[pl.ds(g * 2 * N_PER, 2 * N_PER)])

    return k(src, dst)


def _sc_gather_rows(x, srcarr):
    EPW = E // 32
    CH = 200
    mesh = plsc.VectorSubcoreMesh(core_axis_name="c", subcore_axis_name="s")

    @functools.partial(
        pl.kernel,
        mesh=mesh,
        out_type=jax.ShapeDtypeStruct((E, D), jnp.float32),
        scratch_types=[
            pltpu.VMEM((EPW,), jnp.int32),
            pltpu.VMEM((CH, D), jnp.float32),
            pltpu.SemaphoreType.DMA,
        ],
        compiler_params=_SC_PARAMS,
    )
    def k(x_hbm, src_hbm, out_hbm, idx_v, rows_v, sem):
        c = lax.axis_index("c")
        s = lax.axis_index("s")
        wid = s * 2 + c
        base = wid * EPW
        pltpu.sync_copy(src_hbm.at[pl.ds(base, EPW)], idx_v)

        def body(ci, _):
            pltpu.async_copy(x_hbm.at[idx_v.at[pl.ds(ci * CH, CH)]], rows_v, sem).wait()
            pltpu.sync_copy(rows_v, out_hbm.at[pl.ds(base + ci * CH, CH)])
            return 0

        lax.fori_loop(0, EPW // CH, body, 0)

    return k(x, srcarr)


def _sc_build_bias(src, dst, ef, nm):
    mesh = plsc.VectorSubcoreMesh(core_axis_name="c", subcore_axis_name="s")

    @functools.partial(
        pl.kernel,
        mesh=mesh,
        out_type=jax.ShapeDtypeStruct((NC,), jnp.float32),
        scratch_types=[
            pltpu.VMEM((KK,), jnp.float32),
            pltpu.VMEM((E_PER,), jnp.int32),
            pltpu.VMEM((E_PER,), jnp.int32),
            pltpu.VMEM((E_PER,), jnp.float32),
            pltpu.VMEM((N_PER,), jnp.int32),
        ],
        compiler_params=_SC_PARAMS,
    )
    def k(src_hbm, dst_hbm, ef_hbm, nm_hbm, bias_hbm, bias_buf, s_buf, d_buf, e_buf, nm_buf):
        c = lax.axis_index("c")
        s = lax.axis_index("s")
        wid = s * 2 + c
        zero16f = jnp.zeros((16,), jnp.float32)

        @pl.when(wid < B)
        def _():
            g = wid
            pltpu.sync_copy(src_hbm.at[pl.ds(g * E_PER, E_PER)], s_buf)
            pltpu.sync_copy(dst_hbm.at[pl.ds(g * E_PER, E_PER)], d_buf)
            pltpu.sync_copy(ef_hbm.at[pl.ds(g * E_PER, E_PER)], e_buf)
            pltpu.sync_copy(nm_hbm.at[pl.ds(g * N_PER, N_PER)], nm_buf)

            def zbody(i, _):
                bias_buf[pl.ds(i * 16, 16)] = zero16f
                return 0

            lax.fori_loop(0, KK // 16, zbody, 0)

            goff = g * N_PER

            def body(i, _):
                sl = s_buf[pl.ds(i * 16, 16)] - goff
                dl = d_buf[pl.ds(i * 16, 16)] - goff
                ev = e_buf[pl.ds(i * 16, 16)]
                ms = plsc.load_gather(nm_buf, [sl])
                md = plsc.load_gather(nm_buf, [dl])
                valid = (ms >= 0) & (md >= 0)
                idx = jnp.where(valid, ms * K + md, 0)
                plsc.addupdate_scatter(bias_buf, [idx], ev, mask=valid)
                return 0

            lax.fori_loop(0, E_PER // 16, body, 0)
            pltpu.sync_copy(bias_buf, bias_hbm.at[pl.ds(g * KK, KK)])

    return k(src, dst, ef, nm)


def _dot(x, y, dims):
    return lax.dot_general(
        x, y, (dims, ((), ())),
        precision=lax.Precision.HIGHEST, preferred_element_type=jnp.float32,
    )


def _tc_topk(score3, feat_r, att2):
    def body(s_ref, f_ref, att_ref, fp_ref, perm_ref, nm_ref, av_ref, bv_ref):
        g = pl.program_id(0)
        fg = f_ref[0]
        score_col = s_ref[0]
        ii = lax.broadcasted_iota(jnp.int32, (N_PER, N_PER), 0)
        jj = lax.broadcasted_iota(jnp.int32, (N_PER, N_PER), 1)
        eye_n = jnp.where(ii == jj, 1.0, 0.0)
        score_row = _dot(score_col, eye_n, ((0,), (0,)))
        gt = score_row > score_col
        eq = score_row == score_col
        cmp_f = jnp.where(gt | (eq & (jj < ii)), 1.0, 0.0)
        ones_col = jnp.ones((N_PER, 1), jnp.float32)
        rank_col = _dot(cmp_f, ones_col, ((1,), (0,)))
        rank_row = _dot(rank_col, eye_n, ((0,), (0,)))
        rank_i = rank_row.astype(jnp.int32)
        nm_ref[0] = jnp.where(rank_i < K, rank_i, -1)
        kk = lax.broadcasted_iota(jnp.int32, (K, N_PER), 0)
        Ob_f = jnp.where(rank_i == kk, 1.0, 0.0)
        iota_col = lax.broadcasted_iota(jnp.int32, (N_PER, 1), 0).astype(jnp.float32)
        order_col = _dot(Ob_f, iota_col, ((1,), (0,)))
        ik = lax.broadcasted_iota(jnp.int32, (K, K), 0)
        jk = lax.broadcasted_iota(jnp.int32, (K, K), 1)
        eye_k = jnp.where(ik == jk, 1.0, 0.0)
        order_row = _dot(order_col, eye_k, ((0,), (0,)))
        perm_ref[0] = order_row.astype(jnp.int32) + g * N_PER
        fp_ref[0] = _dot(Ob_f, fg, ((1,), (0,)))
        sl_col = _dot(fg, att_ref[0:1, :], ((1,), (1,)))
        sr_col = _dot(fg, att_ref[1:2, :], ((1,), (1,)))
        av_ref[0] = _dot(Ob_f, sl_col, ((1,), (0,)))
        b_col = _dot(Ob_f, sr_col, ((1,), (0,)))
        bv_ref[0] = _dot(b_col, eye_k, ((0,), (0,)))

    return pl.pallas_call(
        body,
        grid=(B,),
        in_specs=[
            pl.BlockSpec((1, N_PER, 1), lambda g: (g, 0, 0)),
            pl.BlockSpec((1, N_PER, D), lambda g: (g, 0, 0)),
            pl.BlockSpec((2, D), lambda g: (0, 0)),
        ],
        out_specs=[
            pl.BlockSpec((1, K, D), lambda g: (g, 0, 0)),
            pl.BlockSpec((1, 1, K), lambda g: (g, 0, 0)),
            pl.BlockSpec((1, 1, N_PER), lambda g: (g, 0, 0)),
            pl.BlockSpec((1, K, 1), lambda g: (g, 0, 0)),
            pl.BlockSpec((1, 1, K), lambda g: (g, 0, 0)),
        ],
        out_shape=[
            jax.ShapeDtypeStruct((B, K, D), jnp.float32),
            jax.ShapeDtypeStruct((B, 1, K), jnp.int32),
            jax.ShapeDtypeStruct((B, 1, N_PER), jnp.int32),
            jax.ShapeDtypeStruct((B, K, 1), jnp.float32),
            jax.ShapeDtypeStruct((B, 1, K), jnp.float32),
        ],
    )(score3, feat_r, att2)


def _tc_softmax(a3, b3, bias3):
    def body(a_ref, b_ref, bias_ref, ws_ref, row_ref, col_ref):
        g = pl.program_id(0)
        av = a_ref[0]
        bv = b_ref[0]
        w = av + bv
        w = jnp.where(w >= 0, w, SLOPE * w)
        w = w + bias_ref[0]
        m = jnp.max(w, axis=0, keepdims=True)
        ew = jnp.exp(w - m)
        den = jnp.sum(ew, axis=0, keepdims=True)
        ws_ref[0] = ew / den
        rr = lax.broadcasted_iota(jnp.int32, (K, K), 0)
        cc = lax.broadcasted_iota(jnp.int32, (K, K), 1)
        row_ref[0] = g * K + rr
        col_ref[0] = g * K + cc

    return pl.pallas_call(
        body,
        grid=(B,),
        in_specs=[
            pl.BlockSpec((1, K, 1), lambda g: (g, 0, 0)),
            pl.BlockSpec((1, 1, K), lambda g: (g, 0, 0)),
            pl.BlockSpec((1, K, K), lambda g: (g, 0, 0)),
        ],
        out_specs=[
            pl.BlockSpec((1, K, K), lambda g: (g, 0, 0)),
            pl.BlockSpec((1, K, K), lambda g: (g, 0, 0)),
            pl.BlockSpec((1, K, K), lambda g: (g, 0, 0)),
        ],
        out_shape=[
            jax.ShapeDtypeStruct((B, K, K), jnp.float32),
            jax.ShapeDtypeStruct((B, K, K), jnp.int32),
            jax.ShapeDtypeStruct((B, K, K), jnp.int32),
        ],
    )(a3, b3, bias3)


def kernel(feat, edge_index, e_feat, att):
    src = edge_index[0]
    dst = edge_index[1]
    deg = _sc_degrees(src, dst).reshape(B, 2, N_PER)
    out_deg = jnp.maximum(deg[:, 0].reshape(N), 1.0)
    in_deg = jnp.maximum(deg[:, 1].reshape(N), 1.0)
    src_norm = out_deg ** -0.5
    dst_norm = in_deg ** -0.5
    x = feat * src_norm[:, None]
    msg = _sc_gather_rows(x, src) * e_feat[:, None]
    msg = lax.optimization_barrier(msg)
    agg = jnp.zeros((N, D), feat.dtype).at[dst].add(msg)
    f2 = feat - agg * dst_norm[:, None]
    score = jnp.sum(jnp.abs(f2), axis=1)
    feat_r = feat.reshape(B, N_PER, D)
    att2 = att.reshape(2, D)
    feat_p, perm3, nm3, a3, b3 = _tc_topk(
        score.reshape(B, N_PER, 1), feat_r, att2
    )
    bias_flat = _sc_build_bias(src, dst, e_feat, nm3.reshape(N))
    w3, row3, col3 = _tc_softmax(a3, b3, bias_flat.reshape(B, K, K))
    return (
        feat_p.reshape(PN, D),
        w3.reshape(NC),
        perm3.reshape(PN),
        row3.reshape(NC),
        col3.reshape(NC),
    )


# speedup vs baseline: 1.7999x; 1.0899x over previous
import functools

import jax
import jax.numpy as jnp
from jax import lax
from jax.experimental import pallas as pl
from jax.experimental.pallas import tpu as pltpu
from jax.experimental.pallas import tpu_sc as plsc

B = 25
N_PER = 400
N = B * N_PER
DEG = 16
E = N * DEG
D = 128
K = 320
PN = B * K
KK = K * K
NC = B * KK
E_PER = N_PER * DEG
HALF = N_PER // 2
LAMB = 1.0
SLOPE = 0.2

_SC_PARAMS = pltpu.CompilerParams(needs_layout_passes=False)


def _sc_degrees(src, dst):
    mesh = plsc.VectorSubcoreMesh(core_axis_name="c", subcore_axis_name="s")

    @functools.partial(
        pl.kernel,
        mesh=mesh,
        out_type=jax.ShapeDtypeStruct((B * 2 * N_PER,), jnp.float32),
        scratch_types=[
            pltpu.VMEM((E_PER,), jnp.int32),
            pltpu.VMEM((E_PER,), jnp.int32),
            pltpu.VMEM((2 * N_PER,), jnp.float32),
        ],
        compiler_params=_SC_PARAMS,
    )
    def k(src_hbm, dst_hbm, deg_hbm, s_buf, d_buf, deg_buf):
        c = lax.axis_index("c")
        s = lax.axis_index("s")
        wid = s * 2 + c
        zero16f = jnp.zeros((16,), jnp.float32)
        ones16 = jnp.ones((16,), jnp.float32)

        @pl.when(wid < B)
        def _():
            g = wid
            pltpu.sync_copy(src_hbm.at[pl.ds(g * E_PER, E_PER)], s_buf)
            pltpu.sync_copy(dst_hbm.at[pl.ds(g * E_PER, E_PER)], d_buf)

            def zbody(i, _):
                deg_buf[pl.ds(i * 16, 16)] = zero16f
                return 0

            lax.fori_loop(0, (2 * N_PER) // 16, zbody, 0)

            goff = g * N_PER

            def body(i, _):
                sl = s_buf[pl.ds(i * 16, 16)] - goff
                dl = d_buf[pl.ds(i * 16, 16)] - goff
                plsc.addupdate_scatter(deg_buf, [sl], ones16)
                plsc.addupdate_scatter(deg_buf, [dl + N_PER], ones16)
                return 0

            lax.fori_loop(0, E_PER // 16, body, 0)
            pltpu.sync_copy(deg_buf, deg_hbm.at[pl.ds(g * 2 * N_PER, 2 * N_PER)])

    return k(src, dst)


def _sc_gather_rows(x, srcarr):
    EPW = E // 32
    CH = 200
    mesh = plsc.VectorSubcoreMesh(core_axis_name="c", subcore_axis_name="s")

    @functools.partial(
        pl.kernel,
        mesh=mesh,
        out_type=jax.ShapeDtypeStruct((E, D), jnp.float32),
        scratch_types=[
            pltpu.VMEM((EPW,), jnp.int32),
            pltpu.VMEM((CH, D), jnp.float32),
            pltpu.SemaphoreType.DMA,
        ],
        compiler_params=_SC_PARAMS,
    )
    def k(x_hbm, src_hbm, out_hbm, idx_v, rows_v, sem):
        c = lax.axis_index("c")
        s = lax.axis_index("s")
        wid = s * 2 + c
        base = wid * EPW
        pltpu.sync_copy(src_hbm.at[pl.ds(base, EPW)], idx_v)

        def body(ci, _):
            pltpu.async_copy(x_hbm.at[idx_v.at[pl.ds(ci * CH, CH)]], rows_v, sem).wait()
            pltpu.sync_copy(rows_v, out_hbm.at[pl.ds(base + ci * CH, CH)])
            return 0

        lax.fori_loop(0, EPW // CH, body, 0)

    return k(x, srcarr)


def _sc_build_bias(src, dst, ef, nm):
    mesh = plsc.VectorSubcoreMesh(core_axis_name="c", subcore_axis_name="s")

    @functools.partial(
        pl.kernel,
        mesh=mesh,
        out_type=jax.ShapeDtypeStruct((NC,), jnp.float32),
        scratch_types=[
            pltpu.VMEM((KK,), jnp.float32),
            pltpu.VMEM((E_PER,), jnp.int32),
            pltpu.VMEM((E_PER,), jnp.int32),
            pltpu.VMEM((E_PER,), jnp.float32),
            pltpu.VMEM((N_PER,), jnp.int32),
        ],
        compiler_params=_SC_PARAMS,
    )
    def k(src_hbm, dst_hbm, ef_hbm, nm_hbm, bias_hbm, bias_buf, s_buf, d_buf, e_buf, nm_buf):
        c = lax.axis_index("c")
        s = lax.axis_index("s")
        wid = s * 2 + c
        zero16f = jnp.zeros((16,), jnp.float32)

        @pl.when(wid < B)
        def _():
            g = wid
            pltpu.sync_copy(src_hbm.at[pl.ds(g * E_PER, E_PER)], s_buf)
            pltpu.sync_copy(dst_hbm.at[pl.ds(g * E_PER, E_PER)], d_buf)
            pltpu.sync_copy(ef_hbm.at[pl.ds(g * E_PER, E_PER)], e_buf)
            pltpu.sync_copy(nm_hbm.at[pl.ds(g * N_PER, N_PER)], nm_buf)

            def zbody(i, _):
                bias_buf[pl.ds(i * 16, 16)] = zero16f
                return 0

            lax.fori_loop(0, KK // 16, zbody, 0)

            goff = g * N_PER

            def body(i, _):
                sl = s_buf[pl.ds(i * 16, 16)] - goff
                dl = d_buf[pl.ds(i * 16, 16)] - goff
                ev = e_buf[pl.ds(i * 16, 16)]
                ms = plsc.load_gather(nm_buf, [sl])
                md = plsc.load_gather(nm_buf, [dl])
                valid = (ms >= 0) & (md >= 0)
                idx = jnp.where(valid, ms * K + md, 0)
                plsc.addupdate_scatter(bias_buf, [idx], ev, mask=valid)
                return 0

            lax.fori_loop(0, E_PER // 16, body, 0)
            pltpu.sync_copy(bias_buf, bias_hbm.at[pl.ds(g * KK, KK)])

    return k(src, dst, ef, nm)


def _dot(x, y, dims):
    return lax.dot_general(
        x, y, (dims, ((), ())),
        precision=lax.Precision.HIGHEST, preferred_element_type=jnp.float32,
    )


def _tc_topk(score_c3, score_r3, feat_r, att2):
    def body(sc_ref, sr_ref, f_ref, att_ref, fp_ref, perm_ref, nm_ref, ab_ref):
        g = pl.program_id(0)
        fg = f_ref[0]
        score_col = sc_ref[0]
        score_row = sr_ref[0]
        ii = lax.broadcasted_iota(jnp.int32, (N_PER, N_PER), 0)
        jj = lax.broadcasted_iota(jnp.int32, (N_PER, N_PER), 1)
        gt = score_col > score_row
        eq = score_col == score_row
        cmpT_f = jnp.where(gt | (eq & (ii < jj)), 1.0, 0.0)
        ones_col = jnp.ones((N_PER, 1), jnp.float32)
        rank_row = _dot(ones_col, cmpT_f, ((0,), (0,)))
        rank_i = rank_row.astype(jnp.int32)
        nm_ref[0] = jnp.where(rank_i < K, rank_i, -1)
        kk = lax.broadcasted_iota(jnp.int32, (K, N_PER), 0)
        Ob_f = jnp.where(rank_i == kk, 1.0, 0.0)
        iota_col = lax.broadcasted_iota(jnp.int32, (N_PER, 1), 0).astype(jnp.float32)
        slsr = _dot(fg, att_ref[...], ((1,), (1,)))
        rhs = jnp.concatenate([iota_col, slsr], axis=1)
        small = _dot(Ob_f, rhs, ((1,), (0,)))
        perm_ref[0] = small[:, 0:1].astype(jnp.int32) + g * N_PER
        ab_ref[0] = small[:, 1:3]
        fp_ref[0] = _dot(Ob_f, fg, ((1,), (0,)))

    return pl.pallas_call(
        body,
        grid=(B,),
        in_specs=[
            pl.BlockSpec((1, N_PER, 1), lambda g: (g, 0, 0)),
            pl.BlockSpec((1, 1, N_PER), lambda g: (g, 0, 0)),
            pl.BlockSpec((1, N_PER, D), lambda g: (g, 0, 0)),
            pl.BlockSpec((2, D), lambda g: (0, 0)),
        ],
        out_specs=[
            pl.BlockSpec((1, K, D), lambda g: (g, 0, 0)),
            pl.BlockSpec((1, K, 1), lambda g: (g, 0, 0)),
            pl.BlockSpec((1, 1, N_PER), lambda g: (g, 0, 0)),
            pl.BlockSpec((1, K, 2), lambda g: (g, 0, 0)),
        ],
        out_shape=[
            jax.ShapeDtypeStruct((B, K, D), jnp.float32),
            jax.ShapeDtypeStruct((B, K, 1), jnp.int32),
            jax.ShapeDtypeStruct((B, 1, N_PER), jnp.int32),
            jax.ShapeDtypeStruct((B, K, 2), jnp.float32),
        ],
    )(score_c3, score_r3, feat_r, att2)


def _tc_softmax(a3, b3, bias3):
    def body(a_ref, b_ref, bias_ref, ws_ref, row_ref, col_ref):
        g = pl.program_id(0)
        av = a_ref[0]
        bv = b_ref[0]
        w = av + bv
        w = jnp.where(w >= 0, w, SLOPE * w)
        w = w + bias_ref[0]
        m = jnp.max(w, axis=0, keepdims=True)
        ew = jnp.exp(w - m)
        den = jnp.sum(ew, axis=0, keepdims=True)
        ws_ref[0] = ew / den
        rr = lax.broadcasted_iota(jnp.int32, (K, K), 0)
        cc = lax.broadcasted_iota(jnp.int32, (K, K), 1)
        row_ref[0] = g * K + rr
        col_ref[0] = g * K + cc

    return pl.pallas_call(
        body,
        grid=(B,),
        in_specs=[
            pl.BlockSpec((1, K, 1), lambda g: (g, 0, 0)),
            pl.BlockSpec((1, 1, K), lambda g: (g, 0, 0)),
            pl.BlockSpec((1, K, K), lambda g: (g, 0, 0)),
        ],
        out_specs=[
            pl.BlockSpec((1, K, K), lambda g: (g, 0, 0)),
            pl.BlockSpec((1, K, K), lambda g: (g, 0, 0)),
            pl.BlockSpec((1, K, K), lambda g: (g, 0, 0)),
        ],
        out_shape=[
            jax.ShapeDtypeStruct((B, K, K), jnp.float32),
            jax.ShapeDtypeStruct((B, K, K), jnp.int32),
            jax.ShapeDtypeStruct((B, K, K), jnp.int32),
        ],
    )(a3, b3, bias3)


def kernel(feat, edge_index, e_feat, att):
    src = edge_index[0]
    dst = edge_index[1]
    deg = _sc_degrees(src, dst).reshape(B, 2, N_PER)
    out_deg = jnp.maximum(deg[:, 0].reshape(N), 1.0)
    in_deg = jnp.maximum(deg[:, 1].reshape(N), 1.0)
    src_norm = out_deg ** -0.5
    dst_norm = in_deg ** -0.5
    x = feat * src_norm[:, None]
    msg = _sc_gather_rows(x, src) * e_feat[:, None]
    msg = lax.optimization_barrier(msg)
    agg = jnp.zeros((N, D), feat.dtype).at[dst].add(msg)
    f2 = feat - agg * dst_norm[:, None]
    score = jnp.sum(jnp.abs(f2), axis=1)
    feat_r = feat.reshape(B, N_PER, D)
    att2 = att.reshape(2, D)
    feat_p, perm3, nm3, ab3 = _tc_topk(
        score.reshape(B, N_PER, 1), score.reshape(B, 1, N_PER), feat_r, att2
    )
    bias_flat = _sc_build_bias(src, dst, e_feat, nm3.reshape(N))
    a3 = ab3[:, :, 0:1]
    b3 = ab3[:, :, 1:2].reshape(B, 1, K)
    w3, row3, col3 = _tc_softmax(a3, b3, bias_flat.reshape(B, K, K))
    return (
        feat_p.reshape(PN, D),
        w3.reshape(NC),
        perm3.reshape(PN),
        row3.reshape(NC),
        col3.reshape(NC),
    )


# speedup vs baseline: 1.8526x; 1.0293x over previous
import functools

import jax
import jax.numpy as jnp
from jax import lax
from jax.experimental import pallas as pl
from jax.experimental.pallas import tpu as pltpu
from jax.experimental.pallas import tpu_sc as plsc

B = 25
N_PER = 400
N = B * N_PER
DEG = 16
E = N * DEG
D = 128
K = 320
PN = B * K
KK = K * K
NC = B * KK
E_PER = N_PER * DEG
HALF = N_PER // 2
LAMB = 1.0
SLOPE = 0.2

_SC_PARAMS = pltpu.CompilerParams(needs_layout_passes=False)


def _sc_degrees(src, dst):
    mesh = plsc.VectorSubcoreMesh(core_axis_name="c", subcore_axis_name="s")

    @functools.partial(
        pl.kernel,
        mesh=mesh,
        out_type=jax.ShapeDtypeStruct((B * 2 * N_PER,), jnp.float32),
        scratch_types=[
            pltpu.VMEM((E_PER,), jnp.int32),
            pltpu.VMEM((E_PER,), jnp.int32),
            pltpu.VMEM((2 * N_PER,), jnp.float32),
        ],
        compiler_params=_SC_PARAMS,
    )
    def k(src_hbm, dst_hbm, deg_hbm, s_buf, d_buf, deg_buf):
        c = lax.axis_index("c")
        s = lax.axis_index("s")
        wid = s * 2 + c
        zero16f = jnp.zeros((16,), jnp.float32)
        ones16 = jnp.ones((16,), jnp.float32)

        @pl.when(wid < B)
        def _():
            g = wid
            pltpu.sync_copy(src_hbm.at[pl.ds(g * E_PER, E_PER)], s_buf)
            pltpu.sync_copy(dst_hbm.at[pl.ds(g * E_PER, E_PER)], d_buf)

            def zbody(i, _):
                deg_buf[pl.ds(i * 16, 16)] = zero16f
                return 0

            lax.fori_loop(0, (2 * N_PER) // 16, zbody, 0, unroll=4)

            goff = g * N_PER

            def body(i, _):
                sl = s_buf[pl.ds(i * 16, 16)] - goff
                dl = d_buf[pl.ds(i * 16, 16)] - goff
                plsc.addupdate_scatter(deg_buf, [sl], ones16)
                plsc.addupdate_scatter(deg_buf, [dl + N_PER], ones16)
                return 0

            lax.fori_loop(0, E_PER // 16, body, 0, unroll=4)
            pltpu.sync_copy(deg_buf, deg_hbm.at[pl.ds(g * 2 * N_PER, 2 * N_PER)])

    return k(src, dst)


def _sc_gather_rows(x, srcarr):
    EPW = E // 32
    CH = 200
    mesh = plsc.VectorSubcoreMesh(core_axis_name="c", subcore_axis_name="s")

    NCHUNK = EPW // CH

    @functools.partial(
        pl.kernel,
        mesh=mesh,
        out_type=jax.ShapeDtypeStruct((E, D), jnp.float32),
        scratch_types=[
            pltpu.VMEM((EPW,), jnp.int32),
            pltpu.VMEM((CH, D), jnp.float32),
            pltpu.VMEM((CH, D), jnp.float32),
            pltpu.SemaphoreType.DMA,
            pltpu.SemaphoreType.DMA,
        ],
        compiler_params=_SC_PARAMS,
    )
    def k(x_hbm, src_hbm, out_hbm, idx_v, rows0, rows1, sem0, sem1):
        c = lax.axis_index("c")
        s = lax.axis_index("s")
        wid = s * 2 + c
        base = wid * EPW
        pltpu.sync_copy(src_hbm.at[pl.ds(base, EPW)], idx_v)
        bufs = (rows0, rows1)
        sems = (sem0, sem1)
        handles = [None, None]
        handles[0] = pltpu.async_copy(x_hbm.at[idx_v.at[pl.ds(0, CH)]], rows0, sem0)
        for ci in range(1, NCHUNK):
            b = ci % 2
            handles[b] = pltpu.async_copy(
                x_hbm.at[idx_v.at[pl.ds(ci * CH, CH)]], bufs[b], sems[b]
            )
            handles[1 - b].wait()
            pltpu.sync_copy(bufs[1 - b], out_hbm.at[pl.ds(base + (ci - 1) * CH, CH)])
        last = (NCHUNK - 1) % 2
        handles[last].wait()
        pltpu.sync_copy(bufs[last], out_hbm.at[pl.ds(base + (NCHUNK - 1) * CH, CH)])

    return k(x, srcarr)


def _sc_build_bias(src, dst, ef, nm):
    mesh = plsc.VectorSubcoreMesh(core_axis_name="c", subcore_axis_name="s")

    @functools.partial(
        pl.kernel,
        mesh=mesh,
        out_type=jax.ShapeDtypeStruct((NC,), jnp.float32),
        scratch_types=[
            pltpu.VMEM((KK,), jnp.float32),
            pltpu.VMEM((E_PER,), jnp.int32),
            pltpu.VMEM((E_PER,), jnp.int32),
            pltpu.VMEM((E_PER,), jnp.float32),
            pltpu.VMEM((N_PER,), jnp.int32),
        ],
        compiler_params=_SC_PARAMS,
    )
    def k(src_hbm, dst_hbm, ef_hbm, nm_hbm, bias_hbm, bias_buf, s_buf, d_buf, e_buf, nm_buf):
        c = lax.axis_index("c")
        s = lax.axis_index("s")
        wid = s * 2 + c
        zero16f = jnp.zeros((16,), jnp.float32)

        @pl.when(wid < B)
        def _():
            g = wid
            pltpu.sync_copy(src_hbm.at[pl.ds(g * E_PER, E_PER)], s_buf)
            pltpu.sync_copy(dst_hbm.at[pl.ds(g * E_PER, E_PER)], d_buf)
            pltpu.sync_copy(ef_hbm.at[pl.ds(g * E_PER, E_PER)], e_buf)
            pltpu.sync_copy(nm_hbm.at[pl.ds(g * N_PER, N_PER)], nm_buf)

            def zbody(i, _):
                bias_buf[pl.ds(i * 16, 16)] = zero16f
                return 0

            lax.fori_loop(0, KK // 16, zbody, 0, unroll=8)

            goff = g * N_PER

            def body(i, _):
                sl = s_buf[pl.ds(i * 16, 16)] - goff
                dl = d_buf[pl.ds(i * 16, 16)] - goff
                ev = e_buf[pl.ds(i * 16, 16)]
                ms = plsc.load_gather(nm_buf, [sl])
                md = plsc.load_gather(nm_buf, [dl])
                valid = (ms >= 0) & (md >= 0)
                idx = jnp.where(valid, ms * K + md, 0)
                plsc.addupdate_scatter(bias_buf, [idx], ev, mask=valid)
                return 0

            lax.fori_loop(0, E_PER // 16, body, 0, unroll=4)
            pltpu.sync_copy(bias_buf, bias_hbm.at[pl.ds(g * KK, KK)])

    return k(src, dst, ef, nm)


def _dot(x, y, dims):
    return lax.dot_general(
        x, y, (dims, ((), ())),
        precision=lax.Precision.HIGHEST, preferred_element_type=jnp.float32,
    )


def _tc_topk(score_c3, score_r3, feat_r, att2):
    def body(sc_ref, sr_ref, f_ref, att_ref, fp_ref, perm_ref, nm_ref, ab_ref):
        g = pl.program_id(0)
        fg = f_ref[0]
        score_col = sc_ref[0]
        score_row = sr_ref[0]
        ii = lax.broadcasted_iota(jnp.int32, (N_PER, N_PER), 0)
        jj = lax.broadcasted_iota(jnp.int32, (N_PER, N_PER), 1)
        gt = score_col > score_row
        eq = score_col == score_row
        cmpT_f = jnp.where(gt | (eq & (ii < jj)), 1.0, 0.0)
        ones_col = jnp.ones((N_PER, 1), jnp.float32)
        rank_row = _dot(ones_col, cmpT_f, ((0,), (0,)))
        rank_i = rank_row.astype(jnp.int32)
        nm_ref[0] = jnp.where(rank_i < K, rank_i, -1)
        kk = lax.broadcasted_iota(jnp.int32, (K, N_PER), 0)
        Ob_f = jnp.where(rank_i == kk, 1.0, 0.0)
        iota_col = lax.broadcasted_iota(jnp.int32, (N_PER, 1), 0).astype(jnp.float32)
        slsr = _dot(fg, att_ref[...], ((1,), (1,)))
        rhs = jnp.concatenate([iota_col, slsr], axis=1)
        small = _dot(Ob_f, rhs, ((1,), (0,)))
        perm_ref[0] = small[:, 0:1].astype(jnp.int32) + g * N_PER
        ab_ref[0] = small[:, 1:3]
        fp_ref[0] = _dot(Ob_f, fg, ((1,), (0,)))

    return pl.pallas_call(
        body,
        grid=(B,),
        in_specs=[
            pl.BlockSpec((1, N_PER, 1), lambda g: (g, 0, 0)),
            pl.BlockSpec((1, 1, N_PER), lambda g: (g, 0, 0)),
            pl.BlockSpec((1, N_PER, D), lambda g: (g, 0, 0)),
            pl.BlockSpec((2, D), lambda g: (0, 0)),
        ],
        out_specs=[
            pl.BlockSpec((1, K, D), lambda g: (g, 0, 0)),
            pl.BlockSpec((1, K, 1), lambda g: (g, 0, 0)),
            pl.BlockSpec((1, 1, N_PER), lambda g: (g, 0, 0)),
            pl.BlockSpec((1, K, 2), lambda g: (g, 0, 0)),
        ],
        out_shape=[
            jax.ShapeDtypeStruct((B, K, D), jnp.float32),
            jax.ShapeDtypeStruct((B, K, 1), jnp.int32),
            jax.ShapeDtypeStruct((B, 1, N_PER), jnp.int32),
            jax.ShapeDtypeStruct((B, K, 2), jnp.float32),
        ],
    )(score_c3, score_r3, feat_r, att2)


def _tc_softmax(a3, b3, bias3):
    def body(a_ref, b_ref, bias_ref, ws_ref, row_ref, col_ref):
        g = pl.program_id(0)
        av = a_ref[0]
        bv = b_ref[0]
        w = av + bv
        w = jnp.where(w >= 0, w, SLOPE * w)
        w = w + bias_ref[0]
        m = jnp.max(w, axis=0, keepdims=True)
        ew = jnp.exp(w - m)
        den = jnp.sum(ew, axis=0, keepdims=True)
        ws_ref[0] = ew / den
        rr = lax.broadcasted_iota(jnp.int32, (K, K), 0)
        cc = lax.broadcasted_iota(jnp.int32, (K, K), 1)
        row_ref[0] = g * K + rr
        col_ref[0] = g * K + cc

    return pl.pallas_call(
        body,
        grid=(B,),
        in_specs=[
            pl.BlockSpec((1, K, 1), lambda g: (g, 0, 0)),
            pl.BlockSpec((1, 1, K), lambda g: (g, 0, 0)),
            pl.BlockSpec((1, K, K), lambda g: (g, 0, 0)),
        ],
        out_specs=[
            pl.BlockSpec((1, K, K), lambda g: (g, 0, 0)),
            pl.BlockSpec((1, K, K), lambda g: (g, 0, 0)),
            pl.BlockSpec((1, K, K), lambda g: (g, 0, 0)),
        ],
        out_shape=[
            jax.ShapeDtypeStruct((B, K, K), jnp.float32),
            jax.ShapeDtypeStruct((B, K, K), jnp.int32),
            jax.ShapeDtypeStruct((B, K, K), jnp.int32),
        ],
    )(a3, b3, bias3)


def kernel(feat, edge_index, e_feat, att):
    src = edge_index[0]
    dst = edge_index[1]
    deg = _sc_degrees(src, dst).reshape(B, 2, N_PER)
    out_deg = jnp.maximum(deg[:, 0].reshape(N), 1.0)
    in_deg = jnp.maximum(deg[:, 1].reshape(N), 1.0)
    src_norm = out_deg ** -0.5
    dst_norm = in_deg ** -0.5
    x = feat * src_norm[:, None]
    msg = _sc_gather_rows(x, src) * e_feat[:, None]
    msg = lax.optimization_barrier(msg)
    agg = jnp.zeros((N, D), feat.dtype).at[dst].add(msg)
    f2 = feat - agg * dst_norm[:, None]
    score = jnp.sum(jnp.abs(f2), axis=1)
    feat_r = feat.reshape(B, N_PER, D)
    att2 = att.reshape(2, D)
    feat_p, perm3, nm3, ab3 = _tc_topk(
        score.reshape(B, N_PER, 1), score.reshape(B, 1, N_PER), feat_r, att2
    )
    bias_flat = _sc_build_bias(src, dst, e_feat, nm3.reshape(N))
    a3 = ab3[:, :, 0:1]
    b3 = ab3[:, :, 1:2].reshape(B, 1, K)
    w3, row3, col3 = _tc_softmax(a3, b3, bias_flat.reshape(B, K, K))
    return (
        feat_p.reshape(PN, D),
        w3.reshape(NC),
        perm3.reshape(PN),
        row3.reshape(NC),
        col3.reshape(NC),
    )
